# baseline (speedup 1.0000x reference)
import jax
import jax.numpy as jnp
from jax.experimental import pallas as pl
from jax.experimental.pallas import tpu as pltpu

_T, _B, _OBJ, _FEAT, _R, _EMB, _NBL, _MP = 2, 8, 128, 64, 3, 16, 2, 2
_PREC = jax.lax.Precision.HIGHEST
_G = _T * _B
_GPB = 8
_NPB = _GPB * _OBJ


def _gnn_body(adj_ref, unary_ref, We_ref, be_ref, Wr_ref, Wrel_ref, bg_ref,
              Wd_ref, bd_ref, Wb_ref, bb_ref, out_ref):
    f32 = jnp.float32

    an = []
    for k in range(_GPB):
        blocks = []
        for r in range(_R):
            a = (adj_ref[k, r] != 0).astype(f32)
            deg = jnp.sum(a, axis=0, keepdims=True)
            blocks.append(a * (1.0 / jnp.maximum(deg, 1.0)))
        an.append(jnp.concatenate(blocks, axis=0))

    x = jnp.concatenate(
        [jax.lax.dot_general(unary_ref[k], We_ref[...],
                             (((0,), (1,)), ((), ())),
                             preferred_element_type=f32, precision=_PREC)
         for k in range(_GPB)], axis=0) + be_ref[...]

    for l in range(_NBL):
        w_root = Wr_ref[l]
        b = bg_ref[l:l + 1, :]
        for _ in range(_MP):
            t = [jnp.dot(x, Wrel_ref[l, r], preferred_element_type=f32, precision=_PREC)
                 for r in range(_R)]
            root = jnp.dot(x, w_root, preferred_element_type=f32, precision=_PREC)
            aggs = []
            for k in range(_GPB):
                sl = slice(k * _OBJ, (k + 1) * _OBJ)
                tk = jnp.concatenate([t[r][sl] for r in range(_R)],
                                     axis=0)
                aggs.append(jax.lax.dot_general(
                    an[k], tk, (((0,), (0,)), ((), ())),
                    preferred_element_type=f32, precision=_PREC))
            x = jnp.maximum(jnp.concatenate(aggs, axis=0) + root + b, 0.0)

    pooled = jnp.concatenate(
        [jnp.max(x[k * _OBJ:(k + 1) * _OBJ], axis=0, keepdims=True)
         for k in range(_GPB)], axis=0)
    h = jnp.maximum(jnp.dot(pooled, Wd_ref[...],
                            preferred_element_type=f32, precision=_PREC) + bd_ref[...], 0.0)
    val = jnp.sum(h * Wb_ref[...], axis=1,
                  keepdims=True) + bb_ref[...]

    gi = jax.lax.broadcasted_iota(jnp.int32, (_GPB, _GPB), 0)
    bi = jax.lax.broadcasted_iota(jnp.int32, (_GPB, _GPB), 1)
    row = jnp.sum(jnp.where(gi == bi, val, 0.0), axis=0,
                  keepdims=True)
    j = pl.program_id(0)
    out_ref[pl.ds(j, 1), :] = row


def kernel(unary_tensor, binary_tensor, W_embed, b_embed, W_root, W_rel,
           b_gnn, W_d, b_d, W_b, b_b):
    adj = binary_tensor.reshape(_G, _OBJ, _OBJ, _R).transpose(0, 3, 1, 2)
    unary = jnp.swapaxes(unary_tensor.astype(jnp.float32), 2, 3).reshape(
        _G, _FEAT, _OBJ)
    wet = W_embed.T
    wb = W_b.reshape(1, 128)
    be = b_embed.reshape(1, _EMB)
    bd = b_d.reshape(1, 128)
    bb = b_b.reshape(1, 1)

    full = lambda *shape: pl.BlockSpec(shape, lambda g: (0,) * len(shape))
    return pl.pallas_call(
        _gnn_body,
        grid=(_G // _GPB,),
        in_specs=[
            pl.BlockSpec((_GPB, _R, _OBJ, _OBJ), lambda g: (g, 0, 0, 0)),
            pl.BlockSpec((_GPB, _FEAT, _OBJ), lambda g: (g, 0, 0)),
            full(_EMB, _FEAT),
            full(1, _EMB),
            full(_NBL, _EMB, _EMB),
            full(_NBL, _R, _EMB, _EMB),
            full(_NBL, _EMB),
            full(_EMB, 128),
            full(1, 128),
            full(1, 128),
            full(1, 1),
        ],
        out_specs=pl.BlockSpec((_T, _B), lambda g: (0, 0)),
        out_shape=jax.ShapeDtypeStruct((_T, _B), jnp.float32),
        compiler_params=pltpu.CompilerParams(
            dimension_semantics=("arbitrary",)),
    )(adj, unary, wet, be, W_root, W_rel, b_gnn, W_d, bd, wb, bb)


# speedup vs baseline: 1.6827x; 1.6827x over previous
import jax
import jax.numpy as jnp
from jax.experimental import pallas as pl
from jax.experimental.pallas import tpu as pltpu

_T, _B, _OBJ, _FEAT, _R, _EMB, _NBL, _MP = 2, 8, 128, 64, 3, 16, 2, 2
_G = _T * _B
_GPB = 8
_NPB = _GPB * _OBJ

_BF = jnp.bfloat16


def _split3(a):
    f32 = jnp.float32
    hi = a.astype(_BF)
    r1 = a - hi.astype(f32)
    mid = r1.astype(_BF)
    lo = (r1 - mid.astype(f32)).astype(_BF)
    return hi, mid, lo


def _dot6(a, b, dims):
    f32 = jnp.float32
    ah, am, al = _split3(a)
    bh, bm, bl = _split3(b)
    d = lambda u, v: jax.lax.dot_general(u, v, dims,
                                         preferred_element_type=f32)
    return (d(ah, bh) + (d(ah, bm) + d(am, bh))
            + (d(ah, bl) + d(am, bm) + d(al, bh)))


def _mm6(a, b):
    return _dot6(a, b, (((a.ndim - 1,), (0,)), ((), ())))


def _dot3(a, b, dims):
    f32 = jnp.float32
    return jax.lax.dot_general(a.astype(_BF), b.astype(_BF), dims,
                               preferred_element_type=f32)


def _mm3(a, b):
    return _dot3(a, b, (((a.ndim - 1,), (0,)), ((), ())))


def _gnn_body(adj_ref, unary_ref, We_ref, be_ref, Wr_ref, Wrel_ref, bg_ref,
              Wd_ref, bd_ref, Wb_ref, bb_ref, out_ref):
    f32 = jnp.float32

    an = []
    for k in range(_GPB):
        blocks = []
        for r in range(_R):
            a = (adj_ref[k, r] != 0).astype(f32)
            deg = jnp.sum(a, axis=0, keepdims=True)
            blocks.append(a * (1.0 / jnp.maximum(deg, 1.0)))
        an.append(jnp.concatenate(blocks, axis=0))

    x = jnp.concatenate(
        [_dot3(unary_ref[k], We_ref[...], (((0,), (1,)), ((), ())))
         for k in range(_GPB)], axis=0) + be_ref[...]

    for l in range(_NBL):
        w_root = Wr_ref[l]
        b = bg_ref[l:l + 1, :]
        for _ in range(_MP):
            t = [_mm3(x, Wrel_ref[l, r])
                 for r in range(_R)]
            root = _mm3(x, w_root)
            aggs = []
            for k in range(_GPB):
                sl = slice(k * _OBJ, (k + 1) * _OBJ)
                tk = jnp.concatenate([t[r][sl] for r in range(_R)],
                                     axis=0)
                aggs.append(_dot6(
                    an[k], tk, (((0,), (0,)), ((), ()))))
            x = jnp.maximum(jnp.concatenate(aggs, axis=0) + root + b, 0.0)

    pooled = jnp.concatenate(
        [jnp.max(x[k * _OBJ:(k + 1) * _OBJ], axis=0, keepdims=True)
         for k in range(_GPB)], axis=0)
    h = jnp.maximum(_mm3(pooled, Wd_ref[...]) + bd_ref[...], 0.0)
    prods = h.astype(_BF).astype(f32) * Wb_ref[...].astype(_BF).astype(f32)
    val = jnp.sum(prods, axis=1, keepdims=True) + bb_ref[...]

    gi = jax.lax.broadcasted_iota(jnp.int32, (_GPB, _GPB), 0)
    bi = jax.lax.broadcasted_iota(jnp.int32, (_GPB, _GPB), 1)
    row = jnp.sum(jnp.where(gi == bi, val, 0.0), axis=0,
                  keepdims=True)
    j = pl.program_id(0)
    out_ref[pl.ds(j, 1), :] = row


def kernel(unary_tensor, binary_tensor, W_embed, b_embed, W_root, W_rel,
           b_gnn, W_d, b_d, W_b, b_b):
    adj = binary_tensor.reshape(_G, _OBJ, _OBJ, _R).transpose(0, 3, 1, 2)
    unary = jnp.swapaxes(unary_tensor.astype(jnp.float32), 2, 3).reshape(
        _G, _FEAT, _OBJ)
    wet = W_embed.T
    wb = W_b.reshape(1, 128)
    be = b_embed.reshape(1, _EMB)
    bd = b_d.reshape(1, 128)
    bb = b_b.reshape(1, 1)

    full = lambda *shape: pl.BlockSpec(shape, lambda g: (0,) * len(shape))
    return pl.pallas_call(
        _gnn_body,
        grid=(_G // _GPB,),
        in_specs=[
            pl.BlockSpec((_GPB, _R, _OBJ, _OBJ), lambda g: (g, 0, 0, 0)),
            pl.BlockSpec((_GPB, _FEAT, _OBJ), lambda g: (g, 0, 0)),
            full(_EMB, _FEAT),
            full(1, _EMB),
            full(_NBL, _EMB, _EMB),
            full(_NBL, _R, _EMB, _EMB),
            full(_NBL, _EMB),
            full(_EMB, 128),
            full(1, 128),
            full(1, 128),
            full(1, 1),
        ],
        out_specs=pl.BlockSpec((_T, _B), lambda g: (0, 0)),
        out_shape=jax.ShapeDtypeStruct((_T, _B), jnp.float32),
        compiler_params=pltpu.CompilerParams(
            dimension_semantics=("arbitrary",)),
    )(adj, unary, wet, be, W_root, W_rel, b_gnn, W_d, bd, wb, bb)
